# baseline (speedup 1.0000x reference)
import jax
import jax.numpy as jnp
from jax import lax
from jax.experimental import pallas as pl
from jax.experimental.pallas import tpu as pltpu
from jax.experimental.pallas import tpu_sc as plsc

D = 32
B = 16384
L = 16
NC, NS = 2, 16
NW = NC * NS
BPW = B // NW
Q = 128
NQ = BPW // Q
GPR = 128 // D
B_GROUPS = 1000000 * D // 128


def _sc_kernel(w_hbm, n_hbm, idx_hbm, out_hbm, idx_v, gidx_v, buf_a, buf_b,
               acc, sem_a, sem_b):
    wid = lax.axis_index("s") * NC + lax.axis_index("c")
    base = wid * BPW
    pltpu.sync_copy(idx_hbm.at[pl.ds(base, BPW)], idx_v)

    for c in range(BPW // L):
        sl = pl.ds(c * L, L)
        gidx_v[sl] = lax.shift_right_logical(idx_v[sl], 2)

    iota16 = lax.iota(jnp.int32, L)

    for q in range(NQ):
        cp_a = pltpu.async_copy(w_hbm.at[gidx_v.at[pl.ds(q * Q, Q)]], buf_a,
                                sem_a)
        cp_b = pltpu.async_copy(n_hbm.at[gidx_v.at[pl.ds(q * Q, Q)]], buf_b,
                                sem_b)
        cp_a.wait()
        cp_b.wait()

        def ext(r0, carry):
            rows16 = iota16 + r0 * L
            idxc = idx_v[pl.ds(q * Q + r0 * L, L)]
            off = lax.shift_left(jnp.bitwise_and(idxc, GPR - 1), 5)
            accrows = rows16 + q * Q
            for j in range(D):
                a = plsc.load_gather(buf_a, [rows16, off + j])
                b = plsc.load_gather(buf_b, [rows16, off + j])
                jv = jnp.full((L,), j, jnp.int32)
                plsc.store_scatter(acc, [accrows, jv], a + b)
            return carry

        lax.fori_loop(0, Q // L, ext, 0)

    pltpu.sync_copy(acc, out_hbm.at[pl.ds(base, BPW)])


@jax.jit
def _run(weight, name_emb, idx):
    fn = pl.kernel(
        _sc_kernel,
        out_type=jax.ShapeDtypeStruct((B, D), jnp.float32),
        mesh=plsc.VectorSubcoreMesh(core_axis_name="c", subcore_axis_name="s"),
        compiler_params=pltpu.CompilerParams(needs_layout_passes=False),
        scratch_types=[
            pltpu.VMEM((BPW,), jnp.int32),
            pltpu.VMEM((BPW,), jnp.int32),
            pltpu.VMEM((Q, 128), jnp.float32),
            pltpu.VMEM((Q, 128), jnp.float32),
            pltpu.VMEM((BPW, D), jnp.float32),
            pltpu.SemaphoreType.DMA,
            pltpu.SemaphoreType.DMA,
        ],
    )
    return fn(weight.reshape(B_GROUPS, 128), name_emb.reshape(B_GROUPS, 128),
              idx)


def kernel(weight, name_emb, idx):
    return _run(weight, name_emb, idx.astype(jnp.int32))
